# baseline (speedup 1.0000x reference)
import functools

import jax
import jax.numpy as jnp
from jax import lax
from jax.experimental import pallas as pl
from jax.experimental.pallas import tpu as pltpu
from jax.experimental.pallas import tpu_sc as plsc

D = 65536
K = 32768

_info = plsc.get_sparse_core_info()
_NC, _NS, _L = _info.num_cores, _info.num_subcores, _info.num_lanes
_NW = _NC * _NS
_PER_W = K // _NW
_NCH = 8
_CH = _PER_W // _NCH


def _zscore_body(x_hbm, ids_hbm, sb_hbm, out_hbm, idx_v, xg, sbg, og,
                 *sems):
    wid = lax.axis_index("s") * _NC + lax.axis_index("c")
    base = wid * _PER_W

    idx_cp = [
        pltpu.async_copy(ids_hbm.at[pl.ds(base + h * (_PER_W // 2),
                                          _PER_W // 2)],
                         idx_v.at[pl.ds(h * (_PER_W // 2), _PER_W // 2)],
                         sems[_NCH + h])
        for h in range(2)
    ]

    copies = []
    for h in range(_NCH):
        if h == 0:
            idx_cp[0].wait()
        if h == _NCH // 2:
            idx_cp[1].wait()
        hs = pl.ds(h * _CH, _CH)
        idx_h = idx_v.at[hs]
        sem = sems[h]
        copies.append((pltpu.async_copy(x_hbm.at[idx_h], xg.at[hs], sem),
                       pltpu.async_copy(sb_hbm.at[idx_h], sbg.at[hs], sem)))

    out_cp = []
    for h in range(_NCH):
        for cp in copies[h]:
            cp.wait()
        for c in range(h * (_CH // _L), (h + 1) * (_CH // _L)):
            sl = pl.ds(c * _L, _L)
            w = sbg[sl]
            sv = lax.bitcast_convert_type(w << 16, jnp.float32)
            bv = lax.bitcast_convert_type(w & jnp.int32(-65536), jnp.float32)
            og[sl] = xg[sl] * sv + bv
        hs = pl.ds(h * _CH, _CH)
        out_cp.append(pltpu.async_copy(
            og.at[hs], out_hbm.at[pl.ds(base + h * _CH, _CH)], sems[h]))
    for cp in out_cp:
        cp.wait()


@jax.jit
def _zscore_sc(x, ids, sb):
    mesh = plsc.VectorSubcoreMesh(core_axis_name="c", subcore_axis_name="s")
    f = functools.partial(
        pl.kernel,
        mesh=mesh,
        out_type=jax.ShapeDtypeStruct((K,), jnp.float32),
        scratch_types=[
            pltpu.VMEM((_PER_W,), jnp.int32),
            pltpu.VMEM((_PER_W,), jnp.float32),
            pltpu.VMEM((_PER_W,), jnp.int32),
            pltpu.VMEM((_PER_W,), jnp.float32),
        ] + [pltpu.SemaphoreType.DMA] * (_NCH + 2),
    )(_zscore_body)
    return f(x, ids, sb)


def kernel(x, neuron_ids, s, b):
    sb = jax.lax.bitcast_convert_type(
        jnp.stack([s.astype(jnp.bfloat16), b.astype(jnp.bfloat16)], axis=-1),
        jnp.int32)
    return _zscore_sc(x, neuron_ids.astype(jnp.int32), sb)
